# Initial kernel scaffold; baseline (speedup 1.0000x reference)
import jax
import jax.numpy as jnp
from jax.experimental import pallas as pl


def kernel(x, edge_index, edge_attr, batch, W_in, b_in, em_w1, em_b1, em_w2, em_b2, root, conv_b, ro_w1, ro_b1, ro_w2, ro_b2):
    raise NotImplementedError("write your pallas kernel here")


# speedup vs baseline: 2.4235x; 2.4235x over previous
import jax
import jax.numpy as jnp
from jax import lax
from jax.experimental import pallas as pl
from jax.experimental.pallas import tpu as pltpu
from jax.experimental.pallas import tpu_sc as plsc

N = 10000
E = 160000
H = 16
G = 64
NC = 2
NS = 16
NW = NC * NS
EPW = E // NW
CH = 125
NCH = EPW // CH
NPC = N // NS

TN = 2000
TE = 1600
TS = 200
SUB = TE // TS

def _mesh():
    return plsc.VectorSubcoreMesh(core_axis_name="c", subcore_axis_name="s")


def _gather_body(h_hbm, idx_hbm, out_hbm, idx_v, rows_v, sem):
    c = lax.axis_index("c")
    s = lax.axis_index("s")
    wid = s * NC + c
    pltpu.sync_copy(idx_hbm.at[pl.ds(wid * NCH, NCH)], idx_v)

    def chunk(j, carry):
        pltpu.async_copy(h_hbm.at[idx_v.at[j]],
                         rows_v.at[pl.ds(j * CH, CH)], sem).wait()
        return carry

    lax.fori_loop(0, NCH, chunk, 0)
    pltpu.sync_copy(rows_v, out_hbm.at[pl.ds(wid * EPW, EPW)])


def _sc_gather(h, idx2d):
    f = pl.kernel(_gather_body, mesh=_mesh(),
                  compiler_params=pltpu.CompilerParams(use_tc_tiling_on_sc=False),
                  out_type=jax.ShapeDtypeStruct((E, H), jnp.float32),
                  scratch_types=[pltpu.VMEM((NCH, CH), jnp.int32),
                                 pltpu.VMEM((EPW, H), jnp.float32),
                                 pltpu.SemaphoreType.DMA])
    return f(h, idx2d)


def _scatter_body(msg_hbm, idx_hbm, zeros_hbm, out_hbm, idx_v, msg_v, agg_sh, sem):
    c = lax.axis_index("c")
    s = lax.axis_index("s")
    wid = s * NC + c
    pltpu.sync_copy(zeros_hbm.at[pl.ds(s * NPC, NPC)],
                    agg_sh.at[pl.ds(s * NPC, NPC)])
    pltpu.sync_copy(idx_hbm.at[pl.ds(wid * NCH, NCH)], idx_v)
    pltpu.sync_copy(msg_hbm.at[pl.ds(wid * EPW, EPW)], msg_v)
    plsc.subcore_barrier()

    def chunk(j, carry):
        pltpu.sync_copy(msg_v.at[pl.ds(j * CH, CH)],
                        agg_sh.at[idx_v.at[j]], add=True)
        return carry

    lax.fori_loop(0, NCH, chunk, 0)
    plsc.subcore_barrier()
    pltpu.sync_copy(agg_sh.at[pl.ds(s * NPC, NPC)],
                    out_hbm.at[c].at[pl.ds(s * NPC, NPC)])


def _sc_scatter(msg, idx2d, zeros_n):
    f = pl.kernel(_scatter_body, mesh=_mesh(),
                  compiler_params=pltpu.CompilerParams(use_tc_tiling_on_sc=False),
                  out_type=jax.ShapeDtypeStruct((NC, N, H), jnp.float32),
                  scratch_types=[pltpu.VMEM((NCH, CH), jnp.int32),
                                 pltpu.VMEM((EPW, H), jnp.float32),
                                 pltpu.VMEM_SHARED((N, H), jnp.float32),
                                 pltpu.SemaphoreType.DMA])
    return f(msg, idx2d, zeros_n)


def _inproj_body(x_ref, w_ref, b_ref, o_ref):
    o_ref[...] = jnp.dot(x_ref[...], w_ref[...],
                         preferred_element_type=jnp.float32) + b_ref[...]


def _inproj(x, w, b_row):
    return pl.pallas_call(
        _inproj_body,
        grid=(N // TN,),
        in_specs=[pl.BlockSpec((TN, x.shape[1]), lambda i: (i, 0)),
                  pl.BlockSpec((x.shape[1], H), lambda i: (0, 0)),
                  pl.BlockSpec((1, H), lambda i: (0, 0))],
        out_specs=pl.BlockSpec((TN, H), lambda i: (i, 0)),
        out_shape=jax.ShapeDtypeStruct((N, H), jnp.float32))(x, w, b_row)


def _msg_body(ea_ref, xj_ref, w1_ref, b1_ref, w2p_ref, b2p_ref, o_ref):
    jj = lax.broadcasted_iota(jnp.int32, (H, H * H), 1)
    ii = lax.broadcasted_iota(jnp.int32, (H, H * H), 0)
    t16 = (jj % H == ii).astype(jnp.float32)
    sj = lax.broadcasted_iota(jnp.int32, (H * H, H), 0)
    so = lax.broadcasted_iota(jnp.int32, (H * H, H), 1)
    smat = (sj // H == so).astype(jnp.float32)
    w1 = w1_ref[...]
    b1 = b1_ref[...]
    w2p = w2p_ref[...]
    b2p = b2p_ref[...]
    for st in range(SUB):
        r0 = st * TS
        ea = ea_ref[r0:r0 + TS, :]
        xj = xj_ref[r0:r0 + TS, :]
        ew = jnp.maximum(
            jnp.dot(ea, w1, preferred_element_type=jnp.float32) + b1, 0.0)
        acc = jnp.zeros((TS, H), jnp.float32)
        for hf in range(2):
            c0 = hf * 128
            wp = jnp.dot(ew, w2p[:, c0:c0 + 128],
                         preferred_element_type=jnp.float32) + b2p[:, c0:c0 + 128]
            xt = jnp.dot(xj, t16[:, c0:c0 + 128],
                         preferred_element_type=jnp.float32)
            acc = acc + jnp.dot(wp * xt, smat[c0:c0 + 128, :],
                                preferred_element_type=jnp.float32)
        o_ref[r0:r0 + TS, :] = acc


def _msg_call(ea, xj, w1, b1_row, w2p, b2p_row):
    return pl.pallas_call(
        _msg_body,
        grid=(E // TE,),
        in_specs=[pl.BlockSpec((TE, H), lambda i: (i, 0)),
                  pl.BlockSpec((TE, H), lambda i: (i, 0)),
                  pl.BlockSpec((H, 2 * H), lambda i: (0, 0)),
                  pl.BlockSpec((1, 2 * H), lambda i: (0, 0)),
                  pl.BlockSpec((2 * H, H * H), lambda i: (0, 0)),
                  pl.BlockSpec((1, H * H), lambda i: (0, 0))],
        out_specs=pl.BlockSpec((TE, H), lambda i: (i, 0)),
        out_shape=jax.ShapeDtypeStruct((E, H), jnp.float32))(
            ea, xj, w1, b1_row, w2p, b2p_row)


def _update_body(h_ref, a0_ref, a1_ref, r_ref, b_ref, o_ref):
    h = h_ref[...]
    m = (a0_ref[...] + a1_ref[...]
         + jnp.dot(h, r_ref[...], preferred_element_type=jnp.float32)
         + b_ref[...])
    o_ref[...] = jnp.maximum(m, 0.0) + h


def _update(h, a0, a1, rootl, b_row):
    return pl.pallas_call(
        _update_body,
        grid=(N // TN,),
        in_specs=[pl.BlockSpec((TN, H), lambda i: (i, 0)),
                  pl.BlockSpec((TN, H), lambda i: (i, 0)),
                  pl.BlockSpec((TN, H), lambda i: (i, 0)),
                  pl.BlockSpec((H, H), lambda i: (0, 0)),
                  pl.BlockSpec((1, H), lambda i: (0, 0))],
        out_specs=pl.BlockSpec((TN, H), lambda i: (i, 0)),
        out_shape=jax.ShapeDtypeStruct((N, H), jnp.float32))(
            h, a0, a1, rootl, b_row)


def _pool_body(b_ref, h_ref, w1_ref, b1_ref, w2_ref, b2_ref, o_ref, sums, cnts):
    i = pl.program_id(0)

    @pl.when(i == 0)
    def _init():
        sums[...] = jnp.zeros_like(sums)
        cnts[...] = jnp.zeros_like(cnts)

    b_row = b_ref[0]
    gidx = lax.broadcasted_iota(jnp.int32, (G, 1), 0)
    pt = (b_row == gidx).astype(jnp.float32)
    sums[...] += jnp.dot(pt, h_ref[...], preferred_element_type=jnp.float32)
    cnts[...] += jnp.sum(pt, axis=1, keepdims=True)

    @pl.when(i == pl.num_programs(0) - 1)
    def _finish():
        g = sums[...] / jnp.maximum(cnts[...], 1.0)
        r = jnp.maximum(
            jnp.dot(g, w1_ref[...], preferred_element_type=jnp.float32)
            + b1_ref[...], 0.0)
        o_ref[...] = (jnp.dot(r, w2_ref[...], preferred_element_type=jnp.float32)
                      + b2_ref[...])


def _pool(batch3, h, w1, b1_row, w2, b2_row):
    return pl.pallas_call(
        _pool_body,
        grid=(N // TN,),
        in_specs=[pl.BlockSpec((1, 1, TN), lambda i: (i, 0, 0)),
                  pl.BlockSpec((TN, H), lambda i: (i, 0)),
                  pl.BlockSpec((H, H), lambda i: (0, 0)),
                  pl.BlockSpec((1, H), lambda i: (0, 0)),
                  pl.BlockSpec((H, 1), lambda i: (0, 0)),
                  pl.BlockSpec((1, 1), lambda i: (0, 0))],
        out_specs=pl.BlockSpec((G, 1), lambda i: (0, 0)),
        out_shape=jax.ShapeDtypeStruct((G, 1), jnp.float32),
        scratch_shapes=[pltpu.VMEM((G, H), jnp.float32),
                        pltpu.VMEM((G, 1), jnp.float32)])(
            batch3, h, w1, b1_row, w2, b2_row)


def kernel(x, edge_index, edge_attr, batch, W_in, b_in, em_w1, em_b1, em_w2,
           em_b2, root, conv_b, ro_w1, ro_b1, ro_w2, ro_b2):
    src2d = edge_index[0].reshape(E // CH, CH)
    dst2d = edge_index[1].reshape(E // CH, CH)
    batch3 = batch.reshape(N // TN, 1, TN)
    zeros_n = jnp.zeros((N, H), jnp.float32)

    h = _inproj(x, W_in, b_in.reshape(1, H))
    for l in range(em_w1.shape[0]):
        w2p = em_w2[l].reshape(2 * H, H, H).transpose(0, 2, 1).reshape(2 * H, H * H)
        b2p = em_b2[l].reshape(H, H).T.reshape(1, H * H)
        xj = _sc_gather(h, src2d)
        msg = _msg_call(edge_attr, xj, em_w1[l], em_b1[l].reshape(1, 2 * H),
                        w2p, b2p)
        aggp = _sc_scatter(msg, dst2d, zeros_n)
        h = _update(h, aggp[0], aggp[1], root[l], conv_b[l].reshape(1, H))
    return _pool(batch3, h, ro_w1, ro_b1.reshape(1, H),
                 ro_w2, ro_b2.reshape(1, 1))
